# baseline (speedup 1.0000x reference)
import functools

import jax
import jax.numpy as jnp
from jax import lax
from jax.experimental import pallas as pl
from jax.experimental.pallas import tpu as pltpu
from jax.experimental.pallas import tpu_sc as plsc

_GDN = lax.GatherDimensionNumbers(
    offset_dims=(), collapsed_slice_dims=(0,), start_index_map=(0,))


def _vgather(vec, idx):
    return lax.gather(
        vec, idx.reshape(L, 1), dimension_numbers=_GDN, slice_sizes=(1,),
        mode=lax.GatherScatterMode.PROMISE_IN_BOUNDS)


NUM_CORES = 2
NUM_SUBCORES = 16
NW = NUM_CORES * NUM_SUBCORES

T_DIM = 200
B_DIM = 16384
D_DIM = 64
L = 16
BPW = B_DIM // NW
CB = BPW // 128
QB = B_DIM // 4
QCB = QB // 128


def _sc_build():
    mesh = plsc.VectorSubcoreMesh(core_axis_name="c", subcore_axis_name="s")

    @functools.partial(
        pl.kernel,
        mesh=mesh,
        out_type=jax.ShapeDtypeStruct((T_DIM, 8, 128, 8, 128), jnp.float32),
        scratch_types=[
            pltpu.VMEM((3 * D_DIM,), jnp.float32),
            pltpu.VMEM((D_DIM * L,), jnp.float32),
            pltpu.VMEM((QB,), jnp.int32),
            pltpu.VMEM((QB,), jnp.int32),
            pltpu.VMEM((QCB, 8, 128), jnp.float32),
            pltpu.VMEM((QCB, 8, 128), jnp.float32),
            pltpu.SemaphoreType.DMA,
            pltpu.SemaphoreType.DMA,
            pltpu.SemaphoreType.DMA,
            pltpu.SemaphoreType.DMA,
        ],
    )
    def k(labT_hbm, table_hbm, out_hbm, table_v, w_v, lab0, lab1, buf0, buf1,
          sl0, sl1, so0, so1):
        wid = lax.axis_index("s") * NUM_CORES + lax.axis_index("c")
        rr = lax.rem(wid, 8)
        qq = wid // 8
        bbase = qq * QB

        pltpu.sync_copy(table_hbm, table_v)

        lane = lax.iota(jnp.int32, L)
        is1 = lane == 1
        is2 = lane == 2
        zero = jnp.zeros((L,), jnp.float32)
        for j in range(D_DIM // L):
            t1v = table_v[pl.ds(D_DIM + j * L, L)]
            t2v = table_v[pl.ds(2 * D_DIM + j * L, L)]
            for l in range(L):
                d = j * L + l
                a = _vgather(t1v, jnp.full((L,), l, jnp.int32))
                b = _vgather(t2v, jnp.full((L,), l, jnp.int32))
                w_v[pl.ds(d * L, L)] = jnp.where(
                    is1, a, jnp.where(is2, b, zero))

        def lab_start(t, lab, sem):
            pltpu.async_copy(labT_hbm.at[t, pl.ds(bbase, QB)], lab, sem)

        def lab_wait(t, lab, sem):
            pltpu.make_async_copy(
                labT_hbm.at[t, pl.ds(bbase, QB)], lab, sem).wait()

        def out_start(t, buf, sem):
            pltpu.async_copy(
                buf, out_hbm.at[t, rr, pl.ds(qq * QCB, QCB)], sem)

        def out_wait(t, buf, sem):
            pltpu.make_async_copy(
                buf, out_hbm.at[t, rr, pl.ds(qq * QCB, QCB)], sem).wait()

        def build(lab, buf):
            @plsc.parallel_loop(0, QCB * 8, unroll=4)
            def cg_body(cg):
                c = cg // 8
                g = lax.rem(cg, 8)
                labs2 = lab[pl.ds(cg * L, L)].reshape(L, 1)
                for dp in range(8):
                    wd = w_v[pl.ds((rr * 8 + dp) * L, L)]
                    v = lax.gather(
                        wd, labs2, dimension_numbers=_GDN,
                        slice_sizes=(1,),
                        mode=lax.GatherScatterMode.PROMISE_IN_BOUNDS)
                    buf[c, dp, pl.ds(g * L, L)] = v

        lab_start(0, lab0, sl0)
        lab_start(1, lab1, sl1)
        lab_wait(0, lab0, sl0)
        build(lab0, buf0)
        out_start(0, buf0, so0)
        lab_start(2, lab0, sl0)
        lab_wait(1, lab1, sl1)
        build(lab1, buf1)
        out_start(1, buf1, so1)

        def pair_body(p, carry):
            t0 = 2 * p
            lab_wait(t0, lab0, sl0)

            @pl.when(t0 + 1 < T_DIM)
            def _():
                lab_start(t0 + 1, lab1, sl1)

            out_wait(t0 - 2, buf0, so0)
            build(lab0, buf0)
            out_start(t0, buf0, so0)

            @pl.when(t0 + 2 < T_DIM)
            def _():
                lab_start(t0 + 2, lab0, sl0)

            @pl.when(t0 + 1 < T_DIM)
            def _():
                lab_wait(t0 + 1, lab1, sl1)
                out_wait(t0 - 1, buf1, so1)
                build(lab1, buf1)
                out_start(t0 + 1, buf1, so1)

            return carry

        lax.fori_loop(1, T_DIM // 2, pair_body, 0)
        out_wait(T_DIM - 2, buf0, so0)
        out_wait(T_DIM - 1, buf1, so1)

    return k


def kernel(speaker_labels, table):
    labT = speaker_labels.T.astype(jnp.int32)
    y = _sc_build()(labT, table.reshape(-1))
    out = y.transpose(2, 4, 0, 1, 3).reshape(B_DIM, T_DIM, D_DIM)
    return out


# speedup vs baseline: 1.0081x; 1.0081x over previous
import functools

import jax
import jax.numpy as jnp
from jax import lax
from jax.experimental import pallas as pl
from jax.experimental.pallas import tpu as pltpu
from jax.experimental.pallas import tpu_sc as plsc

_GDN = lax.GatherDimensionNumbers(
    offset_dims=(), collapsed_slice_dims=(0,), start_index_map=(0,))


def _vgather(vec, idx):
    return lax.gather(
        vec, idx.reshape(L, 1), dimension_numbers=_GDN, slice_sizes=(1,),
        mode=lax.GatherScatterMode.PROMISE_IN_BOUNDS)


NUM_CORES = 2
NUM_SUBCORES = 16
NW = NUM_CORES * NUM_SUBCORES

T_DIM = 200
B_DIM = 16384
D_DIM = 64
L = 16
BPW = B_DIM // NW
CB = BPW // 128
QB = B_DIM // 4
QCB = QB // 128


def _sc_build():
    mesh = plsc.VectorSubcoreMesh(core_axis_name="c", subcore_axis_name="s")

    @functools.partial(
        pl.kernel,
        mesh=mesh,
        out_type=jax.ShapeDtypeStruct((T_DIM, 8, 128, 8, 128), jnp.float32),
        scratch_types=[
            pltpu.VMEM((3 * D_DIM,), jnp.float32),
            pltpu.VMEM((D_DIM * L,), jnp.float32),
            pltpu.VMEM((QB,), jnp.int32),
            pltpu.VMEM((QB,), jnp.int32),
            pltpu.VMEM((QCB, 8, 128), jnp.float32),
            pltpu.VMEM((QCB, 8, 128), jnp.float32),
            pltpu.SemaphoreType.DMA,
            pltpu.SemaphoreType.DMA,
            pltpu.SemaphoreType.DMA,
            pltpu.SemaphoreType.DMA,
        ],
    )
    def k(labT_hbm, table_hbm, out_hbm, table_v, w_v, lab0, lab1, buf0, buf1,
          sl0, sl1, so0, so1):
        wid = lax.axis_index("s") * NUM_CORES + lax.axis_index("c")
        rr = lax.rem(wid, 8)
        qq = wid // 8
        bbase = qq * QB

        pltpu.sync_copy(table_hbm, table_v)

        lane = lax.iota(jnp.int32, L)
        is1 = lane == 1
        is2 = lane == 2
        zero = jnp.zeros((L,), jnp.float32)
        for j in range(D_DIM // L):
            t1v = table_v[pl.ds(D_DIM + j * L, L)]
            t2v = table_v[pl.ds(2 * D_DIM + j * L, L)]
            for l in range(L):
                d = j * L + l
                a = _vgather(t1v, jnp.full((L,), l, jnp.int32))
                b = _vgather(t2v, jnp.full((L,), l, jnp.int32))
                w_v[pl.ds(d * L, L)] = jnp.where(
                    is1, a, jnp.where(is2, b, zero))

        def lab_start(t, lab, sem):
            pltpu.async_copy(labT_hbm.at[t, pl.ds(bbase, QB)], lab, sem)

        def lab_wait(t, lab, sem):
            pltpu.make_async_copy(
                labT_hbm.at[t, pl.ds(bbase, QB)], lab, sem).wait()

        def out_start(t, buf, sem):
            pltpu.async_copy(
                buf, out_hbm.at[t, rr, pl.ds(qq * QCB, QCB)], sem)

        def out_wait(t, buf, sem):
            pltpu.make_async_copy(
                buf, out_hbm.at[t, rr, pl.ds(qq * QCB, QCB)], sem).wait()

        def build(lab, buf):
            @plsc.parallel_loop(0, QCB * 8, unroll=2)
            def cg_body(cg):
                c = cg // 8
                g = lax.rem(cg, 8)
                labs2 = lab[pl.ds(cg * L, L)].reshape(L, 1)
                for dp in range(8):
                    wd = w_v[pl.ds((rr * 8 + dp) * L, L)]
                    v = lax.gather(
                        wd, labs2, dimension_numbers=_GDN,
                        slice_sizes=(1,),
                        mode=lax.GatherScatterMode.PROMISE_IN_BOUNDS)
                    buf[c, dp, pl.ds(g * L, L)] = v

        lab_start(0, lab0, sl0)
        lab_start(1, lab1, sl1)
        lab_wait(0, lab0, sl0)
        build(lab0, buf0)
        out_start(0, buf0, so0)
        lab_start(2, lab0, sl0)
        lab_wait(1, lab1, sl1)
        build(lab1, buf1)
        out_start(1, buf1, so1)

        def pair_body(p, carry):
            t0 = 2 * p
            lab_wait(t0, lab0, sl0)

            @pl.when(t0 + 1 < T_DIM)
            def _():
                lab_start(t0 + 1, lab1, sl1)

            out_wait(t0 - 2, buf0, so0)
            build(lab0, buf0)
            out_start(t0, buf0, so0)

            @pl.when(t0 + 2 < T_DIM)
            def _():
                lab_start(t0 + 2, lab0, sl0)

            @pl.when(t0 + 1 < T_DIM)
            def _():
                lab_wait(t0 + 1, lab1, sl1)
                out_wait(t0 - 1, buf1, so1)
                build(lab1, buf1)
                out_start(t0 + 1, buf1, so1)

            return carry

        lax.fori_loop(1, T_DIM // 2, pair_body, 0)
        out_wait(T_DIM - 2, buf0, so0)
        out_wait(T_DIM - 1, buf1, so1)

    return k


def kernel(speaker_labels, table):
    labT = speaker_labels.T.astype(jnp.int32)
    y = _sc_build()(labT, table.reshape(-1))
    out = y.transpose(2, 4, 0, 1, 3).reshape(B_DIM, T_DIM, D_DIM)
    return out


# speedup vs baseline: 1.1517x; 1.1424x over previous
import functools

import jax
import jax.numpy as jnp
from jax import lax
from jax.experimental import pallas as pl
from jax.experimental.pallas import tpu as pltpu
from jax.experimental.pallas import tpu_sc as plsc

_GDN = lax.GatherDimensionNumbers(
    offset_dims=(), collapsed_slice_dims=(0,), start_index_map=(0,))


def _vgather(vec, idx):
    return lax.gather(
        vec, idx.reshape(L, 1), dimension_numbers=_GDN, slice_sizes=(1,),
        mode=lax.GatherScatterMode.PROMISE_IN_BOUNDS)


NUM_CORES = 2
NUM_SUBCORES = 16
NW = NUM_CORES * NUM_SUBCORES

T_DIM = 200
B_DIM = 16384
D_DIM = 64
L = 16
EB = B_DIM // 8
ECB = EB // 128
TCH = 8
NTC = T_DIM // TCH


def _sc_build():
    mesh = plsc.VectorSubcoreMesh(core_axis_name="c", subcore_axis_name="s")

    @functools.partial(
        pl.kernel,
        mesh=mesh,
        out_type=jax.ShapeDtypeStruct((T_DIM, 8, 128, 8, 128), jnp.float32),
        scratch_types=[
            pltpu.VMEM((3 * D_DIM,), jnp.float32),
            pltpu.VMEM((D_DIM * L,), jnp.float32),
            pltpu.VMEM((TCH, EB), jnp.int32),
            pltpu.VMEM((TCH, EB), jnp.int32),
            pltpu.VMEM((2, ECB, 8, 128), jnp.float32),
            pltpu.VMEM((2, ECB, 8, 128), jnp.float32),
            pltpu.SemaphoreType.DMA,
            pltpu.SemaphoreType.DMA,
            pltpu.SemaphoreType.DMA,
            pltpu.SemaphoreType.DMA,
        ],
    )
    def k(labT_hbm, table_hbm, out_hbm, table_v, w_v, labA, labB, buf0, buf1,
          sl0, sl1, so0, so1):
        wid = lax.axis_index("s") * NUM_CORES + lax.axis_index("c")
        rp = lax.rem(wid, 4)
        q8 = wid // 4
        bbase = q8 * EB

        pltpu.sync_copy(table_hbm, table_v)

        lane = lax.iota(jnp.int32, L)
        is1 = lane == 1
        is2 = lane == 2
        zero = jnp.zeros((L,), jnp.float32)
        for j in range(D_DIM // L):
            t1v = table_v[pl.ds(D_DIM + j * L, L)]
            t2v = table_v[pl.ds(2 * D_DIM + j * L, L)]
            for l in range(L):
                d = j * L + l
                a = _vgather(t1v, jnp.full((L,), l, jnp.int32))
                b = _vgather(t2v, jnp.full((L,), l, jnp.int32))
                w_v[pl.ds(d * L, L)] = jnp.where(
                    is1, a, jnp.where(is2, b, zero))

        def lab_start(tc, lab, sem):
            pltpu.async_copy(
                labT_hbm.at[pl.ds(tc * TCH, TCH), pl.ds(bbase, EB)], lab, sem)

        def lab_wait(tc, lab, sem):
            pltpu.make_async_copy(
                labT_hbm.at[pl.ds(tc * TCH, TCH), pl.ds(bbase, EB)],
                lab, sem).wait()

        def out_start(t, buf, sem):
            pltpu.async_copy(
                buf, out_hbm.at[t, pl.ds(rp * 2, 2), pl.ds(q8 * ECB, ECB)],
                sem)

        def out_wait(t, buf, sem):
            pltpu.make_async_copy(
                buf, out_hbm.at[t, pl.ds(rp * 2, 2), pl.ds(q8 * ECB, ECB)],
                sem).wait()

        def build(lab, j, buf):
            @plsc.parallel_loop(0, ECB * 8, unroll=2)
            def cg_body(cg):
                c = cg // 8
                g = lax.rem(cg, 8)
                labs2 = lab[j, pl.ds(cg * L, L)].reshape(L, 1)
                for ri in range(2):
                    for dp in range(8):
                        wd = w_v[pl.ds(((rp * 2 + ri) * 8 + dp) * L, L)]
                        v = lax.gather(
                            wd, labs2, dimension_numbers=_GDN,
                            slice_sizes=(1,),
                            mode=lax.GatherScatterMode.PROMISE_IN_BOUNDS)
                        buf[ri, c, dp, pl.ds(g * L, L)] = v

        bufs = (buf0, buf1)
        sos = (so0, so1)

        def chunk(tc, lab, skip_wait_first):
            for j in range(TCH):
                t = tc * TCH + j
                bb, ss = bufs[j % 2], sos[j % 2]
                if not (skip_wait_first and j < 2):
                    out_wait(t - 2, bb, ss)
                build(lab, j, bb)
                out_start(t, bb, ss)

        lab_start(0, labA, sl0)
        lab_wait(0, labA, sl0)
        lab_start(1, labB, sl1)
        chunk(0, labA, True)
        lab_start(2, labA, sl0)

        def pair_body(p, carry):
            tc0 = 2 * p + 1
            lab_wait(tc0, labB, sl1)
            chunk(tc0, labB, False)

            @pl.when(tc0 + 2 < NTC)
            def _():
                lab_start(tc0 + 2, labB, sl1)

            lab_wait(tc0 + 1, labA, sl0)
            chunk(tc0 + 1, labA, False)

            @pl.when(tc0 + 3 < NTC)
            def _():
                lab_start(tc0 + 3, labA, sl0)

            return carry

        lax.fori_loop(0, (NTC - 1) // 2, pair_body, 0)
        out_wait(T_DIM - 2, buf0, so0)
        out_wait(T_DIM - 1, buf1, so1)

    return k


def kernel(speaker_labels, table):
    labT = speaker_labels.T.astype(jnp.int32)
    y = _sc_build()(labT, table.reshape(-1))
    out = y.transpose(2, 4, 0, 1, 3).reshape(B_DIM, T_DIM, D_DIM)
    return out
